# baseline (speedup 1.0000x reference)
import jax
import jax.numpy as jnp
from jax.experimental import pallas as pl


def _pick_bm(n: int, target: int) -> int:
    for bm in (target, 400, 200, 100, 40, 8):
        if bm <= target and n % bm == 0:
            return bm
    return n


def _s1_body(x_ref, w1_ref, s1_ref):
    s1 = jnp.dot(x_ref[...].astype(jnp.bfloat16), w1_ref[...],
                 preferred_element_type=jnp.float32)
    s1_ref[...] = s1.astype(jnp.bfloat16)


def _layer1_body(adj_ref, s1_ref, b1_ref, w2_ref, s2_ref):
    a = adj_ref[...].astype(jnp.bfloat16)
    h = jnp.dot(a, s1_ref[...], preferred_element_type=jnp.float32)
    h = jax.nn.relu(h + b1_ref[...])
    s2 = jnp.dot(h.astype(jnp.bfloat16), w2_ref[...],
                 preferred_element_type=jnp.float32)
    s2_ref[...] = s2.astype(jnp.bfloat16)


def _layer2_body(adj_ref, s2_ref, b2_ref, out_ref):
    a = adj_ref[...].astype(jnp.bfloat16)
    z = jnp.dot(a, s2_ref[...], preferred_element_type=jnp.float32)
    z = z + b2_ref[...]
    m = jnp.max(z, axis=1, keepdims=True)
    z = z - m
    lse = jnp.log(jnp.sum(jnp.exp(z), axis=1, keepdims=True))
    out_ref[...] = z - lse


def kernel(x, adj, W1, b1, W2, b2):
    n, f = x.shape
    h = W1.shape[1]
    c = W2.shape[1]

    bm0 = _pick_bm(n, 2000)
    s1 = pl.pallas_call(
        _s1_body,
        grid=(n // bm0,),
        in_specs=[pl.BlockSpec((bm0, f), lambda i: (i, 0)),
                  pl.BlockSpec((f, h), lambda i: (0, 0))],
        out_specs=pl.BlockSpec((bm0, h), lambda i: (i, 0)),
        out_shape=jax.ShapeDtypeStruct((n, h), jnp.bfloat16),
    )(x, W1.astype(jnp.bfloat16))

    bm = _pick_bm(n, 200)
    s2 = pl.pallas_call(
        _layer1_body,
        grid=(n // bm,),
        in_specs=[pl.BlockSpec((bm, n), lambda i: (i, 0)),
                  pl.BlockSpec((n, h), lambda i: (0, 0)),
                  pl.BlockSpec((1, h), lambda i: (0, 0)),
                  pl.BlockSpec((h, c), lambda i: (0, 0))],
        out_specs=pl.BlockSpec((bm, c), lambda i: (i, 0)),
        out_shape=jax.ShapeDtypeStruct((n, c), jnp.bfloat16),
    )(adj, s1, b1.reshape(1, h), W2.astype(jnp.bfloat16))

    out = pl.pallas_call(
        _layer2_body,
        grid=(n // bm,),
        in_specs=[pl.BlockSpec((bm, n), lambda i: (i, 0)),
                  pl.BlockSpec((n, c), lambda i: (0, 0)),
                  pl.BlockSpec((1, c), lambda i: (0, 0))],
        out_specs=pl.BlockSpec((bm, c), lambda i: (i, 0)),
        out_shape=jax.ShapeDtypeStruct((n, c), jnp.float32),
    )(adj, s2, b2.reshape(1, c))

    return out


# speedup vs baseline: 1.1547x; 1.1547x over previous
import jax
import jax.numpy as jnp
from jax.experimental import pallas as pl


def _s1_body(x_ref, w1_ref, s1_ref):
    s1 = jnp.dot(x_ref[...].astype(jnp.bfloat16), w1_ref[...],
                 preferred_element_type=jnp.float32)
    s1_ref[...] = s1.astype(jnp.bfloat16)


def _layer1_body(adj_ref, s1_ref, b1_ref, w2_ref, s2_ref, adjq_ref):
    a32 = adj_ref[...]
    h = jnp.dot(a32.astype(jnp.bfloat16), s1_ref[...],
                preferred_element_type=jnp.float32)
    h = jax.nn.relu(h + b1_ref[...])
    s2 = jnp.dot(h.astype(jnp.bfloat16), w2_ref[...],
                 preferred_element_type=jnp.float32)
    s2_ref[...] = (s2 * (1.0 / 256.0)).astype(jnp.bfloat16)
    t = a32 * 256.0 + jnp.float32(2.0**23)
    t = jnp.minimum(t, jnp.float32(2.0**23 + 255.0))
    adjq_ref[...] = jax.lax.bitcast_convert_type(t, jnp.uint32).astype(jnp.uint8)


def _layer2_body(adjq_ref, s2_ref, b2_ref, out_ref):
    a = adjq_ref[...].astype(jnp.bfloat16)
    z = jnp.dot(a, s2_ref[...], preferred_element_type=jnp.float32)
    z = z + b2_ref[...]
    m = jnp.max(z, axis=1, keepdims=True)
    z = z - m
    lse = jnp.log(jnp.sum(jnp.exp(z), axis=1, keepdims=True))
    out_ref[...] = z - lse


def kernel(x, adj, W1, b1, W2, b2):
    n, f = x.shape
    h = W1.shape[1]
    c = W2.shape[1]

    bm0 = min(n, 2000)
    s1 = pl.pallas_call(
        _s1_body,
        grid=(pl.cdiv(n, bm0),),
        in_specs=[pl.BlockSpec((bm0, f), lambda i: (i, 0)),
                  pl.BlockSpec((f, h), lambda i: (0, 0))],
        out_specs=pl.BlockSpec((bm0, h), lambda i: (i, 0)),
        out_shape=jax.ShapeDtypeStruct((n, h), jnp.bfloat16),
    )(x, W1.astype(jnp.bfloat16))

    bm1 = min(n, 384)
    s2, adjq = pl.pallas_call(
        _layer1_body,
        grid=(pl.cdiv(n, bm1),),
        in_specs=[pl.BlockSpec((bm1, n), lambda i: (i, 0)),
                  pl.BlockSpec((n, h), lambda i: (0, 0)),
                  pl.BlockSpec((1, h), lambda i: (0, 0)),
                  pl.BlockSpec((h, c), lambda i: (0, 0))],
        out_specs=[pl.BlockSpec((bm1, c), lambda i: (i, 0)),
                   pl.BlockSpec((bm1, n), lambda i: (i, 0))],
        out_shape=[jax.ShapeDtypeStruct((n, c), jnp.bfloat16),
                   jax.ShapeDtypeStruct((n, n), jnp.uint8)],
    )(adj, s1, b1.reshape(1, h), W2.astype(jnp.bfloat16))

    bm2 = min(n, 512)
    out = pl.pallas_call(
        _layer2_body,
        grid=(pl.cdiv(n, bm2),),
        in_specs=[pl.BlockSpec((bm2, n), lambda i: (i, 0)),
                  pl.BlockSpec((n, c), lambda i: (0, 0)),
                  pl.BlockSpec((1, c), lambda i: (0, 0))],
        out_specs=pl.BlockSpec((bm2, c), lambda i: (i, 0)),
        out_shape=jax.ShapeDtypeStruct((n, c), jnp.float32),
    )(adjq, s2, b2.reshape(1, c))

    return out


# speedup vs baseline: 1.1591x; 1.0038x over previous
import functools

import jax
import jax.numpy as jnp
from jax.experimental import pallas as pl
from jax.experimental.pallas import tpu as pltpu


def _fused_a_body(x_ref, w1_ref, adj_ref, b1_ref, w2_ref, b2_ref,
                  adjq_ref, s2q_ref, gamma_ref, delta_ref,
                  s1_ref, s2_ref, *, bm1, n):
    i = pl.program_id(0)
    nsteps = pl.num_programs(0)

    @pl.when(i == 0)
    def _s1():
        s1 = jnp.dot(x_ref[...].astype(jnp.bfloat16), w1_ref[...],
                     preferred_element_type=jnp.float32)
        s1_ref[...] = s1.astype(jnp.bfloat16)

    @pl.when((i >= 1) & (i <= nsteps - 2))
    def _layer1():
        a32 = adj_ref[...]
        h = jnp.dot(a32.astype(jnp.bfloat16), s1_ref[...],
                    preferred_element_type=jnp.float32)
        h = jax.nn.relu(h + b1_ref[...])
        s2 = jnp.dot(h.astype(jnp.bfloat16), w2_ref[...],
                     preferred_element_type=jnp.float32)
        s2_ref[pl.ds((i - 1) * bm1, bm1), :] = s2
        t = a32 * 256.0 + jnp.float32(2.0**23 + 128.0)
        t = jnp.minimum(t, jnp.float32(2.0**23 + 128.0 + 255.0))
        b = jax.lax.bitcast_convert_type(t, jnp.uint32).astype(jnp.uint8)
        adjq_ref[...] = jax.lax.bitcast_convert_type(b, jnp.int8)

    @pl.when(i == nsteps - 1)
    def _s2q():
        s2 = s2_ref[...]
        rows = jax.lax.broadcasted_iota(jnp.int32, s2.shape, 0)
        valid = rows < n
        s2m = jnp.where(valid, s2, 0.0)
        sc = jnp.maximum(jnp.max(jnp.abs(s2m), axis=0, keepdims=True),
                         1e-20) * (1.0 / 127.0)
        q = jnp.floor(s2m * (1.0 / sc) + 0.5)
        q = jnp.clip(q, -127.0, 127.0)
        s2q_ref[...] = q[:n, :].astype(jnp.int8)
        gamma = sc * (1.0 / 256.0)
        gamma_ref[...] = gamma
        colsum = jnp.sum(jnp.where(valid, q, 0.0), axis=0, keepdims=True)
        delta_ref[...] = 128.0 * colsum * gamma + b2_ref[...]


def _layer2_body(adjq_ref, s2q_ref, gamma_ref, delta_ref, out_ref):
    zq = jnp.dot(adjq_ref[...], s2q_ref[...],
                 preferred_element_type=jnp.int32)
    z = zq.astype(jnp.float32) * gamma_ref[...] + delta_ref[...]
    m = jnp.max(z, axis=1, keepdims=True)
    z = z - m
    lse = jnp.log(jnp.sum(jnp.exp(z), axis=1, keepdims=True))
    out_ref[...] = z - lse


def kernel(x, adj, W1, b1, W2, b2):
    n, f = x.shape
    h = W1.shape[1]
    c = W2.shape[1]

    bm1 = min(n, 384)
    nb1 = pl.cdiv(n, bm1)
    nsteps = nb1 + 2

    def adj_idx(i):
        return (jnp.clip(i - 1, 0, nb1 - 1), 0)

    adjq, s2q, gamma, delta = pl.pallas_call(
        functools.partial(_fused_a_body, bm1=bm1, n=n),
        grid=(nsteps,),
        in_specs=[pl.BlockSpec((n, f), lambda i: (0, 0)),
                  pl.BlockSpec((f, h), lambda i: (0, 0)),
                  pl.BlockSpec((bm1, n), adj_idx),
                  pl.BlockSpec((1, h), lambda i: (0, 0)),
                  pl.BlockSpec((h, c), lambda i: (0, 0)),
                  pl.BlockSpec((1, c), lambda i: (0, 0))],
        out_specs=[pl.BlockSpec((bm1, n), adj_idx),
                   pl.BlockSpec((n, c), lambda i: (0, 0)),
                   pl.BlockSpec((1, c), lambda i: (0, 0)),
                   pl.BlockSpec((1, c), lambda i: (0, 0))],
        out_shape=[jax.ShapeDtypeStruct((n, n), jnp.int8),
                   jax.ShapeDtypeStruct((n, c), jnp.int8),
                   jax.ShapeDtypeStruct((1, c), jnp.float32),
                   jax.ShapeDtypeStruct((1, c), jnp.float32)],
        scratch_shapes=[pltpu.VMEM((n, h), jnp.bfloat16),
                        pltpu.VMEM((nb1 * bm1, c), jnp.float32)],
    )(x, W1.astype(jnp.bfloat16), adj, b1.reshape(1, h),
      W2.astype(jnp.bfloat16), b2.reshape(1, c))

    bm2 = min(n, 512)
    out = pl.pallas_call(
        _layer2_body,
        grid=(pl.cdiv(n, bm2),),
        in_specs=[pl.BlockSpec((bm2, n), lambda i: (i, 0)),
                  pl.BlockSpec((n, c), lambda i: (0, 0)),
                  pl.BlockSpec((1, c), lambda i: (0, 0)),
                  pl.BlockSpec((1, c), lambda i: (0, 0))],
        out_specs=pl.BlockSpec((bm2, c), lambda i: (i, 0)),
        out_shape=jax.ShapeDtypeStruct((n, c), jnp.float32),
    )(adjq, s2q, gamma, delta)

    return out


# speedup vs baseline: 1.3118x; 1.1318x over previous
import functools

import jax
import jax.numpy as jnp
from jax.experimental import pallas as pl
from jax.experimental.pallas import tpu as pltpu


def _fused_a_body(x_ref, w1_ref, adj_ref, b1_ref, w2_ref,
                  adjq_ref, s2q_ref,
                  s1_ref, s2_ref, *, bm1, n):
    i = pl.program_id(0)
    nsteps = pl.num_programs(0)

    @pl.when(i == 0)
    def _s1():
        s1 = jnp.dot(x_ref[...].astype(jnp.bfloat16), w1_ref[...],
                     preferred_element_type=jnp.float32)
        s1_ref[...] = s1.astype(jnp.bfloat16)

    @pl.when((i >= 1) & (i <= nsteps - 2))
    def _layer1():
        a32 = adj_ref[...]
        h = jnp.dot(a32.astype(jnp.bfloat16), s1_ref[...],
                    preferred_element_type=jnp.float32)
        h = jax.nn.relu(h + b1_ref[...])
        s2 = jnp.dot(h.astype(jnp.bfloat16), w2_ref[...],
                     preferred_element_type=jnp.float32)
        s2_ref[pl.ds((i - 1) * bm1, bm1), :] = s2
        adjq_ref[...] = a32.astype(jnp.float8_e4m3fn)

    @pl.when(i == nsteps - 1)
    def _s2q():
        s2q_ref[...] = s2_ref[...][:n, :].astype(jnp.float8_e4m3fn)


def _layer2_body(adjq_ref, s2q_ref, b2_ref, out_ref):
    zq = jnp.dot(adjq_ref[...], s2q_ref[...],
                 preferred_element_type=jnp.float32)
    z = zq + b2_ref[...]
    m = jnp.max(z, axis=1, keepdims=True)
    z = z - m
    lse = jnp.log(jnp.sum(jnp.exp(z), axis=1, keepdims=True))
    out_ref[...] = z - lse


def kernel(x, adj, W1, b1, W2, b2):
    n, f = x.shape
    h = W1.shape[1]
    c = W2.shape[1]

    bm1 = min(n, 320)
    nb1 = pl.cdiv(n, bm1)
    nsteps = nb1 + 2

    def adj_idx(i):
        return (jnp.clip(i - 1, 0, nb1 - 1), 0)

    adjq, s2q = pl.pallas_call(
        functools.partial(_fused_a_body, bm1=bm1, n=n),
        grid=(nsteps,),
        in_specs=[pl.BlockSpec((n, f), lambda i: (0, 0)),
                  pl.BlockSpec((f, h), lambda i: (0, 0)),
                  pl.BlockSpec((bm1, n), adj_idx),
                  pl.BlockSpec((1, h), lambda i: (0, 0)),
                  pl.BlockSpec((h, c), lambda i: (0, 0))],
        out_specs=[pl.BlockSpec((bm1, n), adj_idx),
                   pl.BlockSpec((n, c), lambda i: (0, 0))],
        out_shape=[jax.ShapeDtypeStruct((n, n), jnp.float8_e4m3fn),
                   jax.ShapeDtypeStruct((n, c), jnp.float8_e4m3fn)],
        scratch_shapes=[pltpu.VMEM((n, h), jnp.bfloat16),
                        pltpu.VMEM((nb1 * bm1, c), jnp.float32)],
    )(x, W1.astype(jnp.bfloat16), adj, b1.reshape(1, h),
      W2.astype(jnp.bfloat16))

    bm2 = min(n, 1024)
    out = pl.pallas_call(
        _layer2_body,
        grid=(pl.cdiv(n, bm2),),
        in_specs=[pl.BlockSpec((bm2, n), lambda i: (i, 0)),
                  pl.BlockSpec((n, c), lambda i: (0, 0)),
                  pl.BlockSpec((1, c), lambda i: (0, 0))],
        out_specs=pl.BlockSpec((bm2, c), lambda i: (i, 0)),
        out_shape=jax.ShapeDtypeStruct((n, c), jnp.float32),
    )(adjq, s2q, b2.reshape(1, c))

    return out
